# baseline (speedup 1.0000x reference)
import functools

import jax
import jax.numpy as jnp
from jax import lax
from jax.experimental import pallas as pl
from jax.experimental.pallas import tpu as pltpu
from jax.experimental.pallas import tpu_sc as plsc

CH = 128
N_PAD = 10240
NW = 32
C = 128
L = 16
BLK = 256
MASK = jnp.int32(0xFFFF)


def _sc_mesh():
    return plsc.VectorSubcoreMesh(core_axis_name="c", subcore_axis_name="s")


@functools.lru_cache(maxsize=None)
def _deg_call(nch):
    @functools.partial(
        pl.kernel,
        out_type=jax.ShapeDtypeStruct((NW, N_PAD), jnp.float32),
        mesh=_sc_mesh(),
        scratch_types=[
            pltpu.VMEM((nch * C,), jnp.int32),
            pltpu.VMEM((N_PAD,), jnp.float32),
        ],
        compiler_params=pltpu.CompilerParams(needs_layout_passes=False),
    )
    def deg_kernel(edges_hbm, degp_hbm, ebuf, hist):
        cid = lax.axis_index("c")
        sid = lax.axis_index("s")
        wid = sid * 2 + cid
        pltpu.sync_copy(edges_hbm.at[wid], ebuf)
        zeros = jnp.zeros((L,), jnp.float32)
        ones = jnp.ones((L,), jnp.float32)

        def zero_body(i, carry):
            hist[pl.ds(i * L, L)] = zeros
            return carry

        lax.fori_loop(0, N_PAD // L, zero_body, 0)

        def vec_body(j, carry):
            w16 = ebuf[pl.ds(j * L, L)]
            d16 = lax.shift_right_logical(w16, 16)
            plsc.addupdate_scatter(hist, [d16], ones)
            return carry

        lax.fori_loop(0, (nch * C) // L, vec_body, 0)
        pltpu.sync_copy(hist, degp_hbm.at[wid])

    return deg_kernel


@functools.lru_cache(maxsize=None)
def _scatter_call(nch):
    @functools.partial(
        pl.kernel,
        out_type=jax.ShapeDtypeStruct((2, N_PAD, CH), jnp.float32),
        mesh=_sc_mesh(),
        scratch_types=[
            pltpu.VMEM((nch, C), jnp.int32),
            pltpu.VMEM((1, C), jnp.int32),
            pltpu.VMEM((1, C), jnp.int32),
            pltpu.VMEM((1, C), jnp.int32),
            pltpu.VMEM((1, C), jnp.int32),
            pltpu.VMEM((C, CH), jnp.float32),
            pltpu.VMEM((C, CH), jnp.float32),
            pltpu.VMEM_SHARED((N_PAD, CH), jnp.float32),
            pltpu.SemaphoreType.DMA,
            pltpu.SemaphoreType.DMA,
        ],
        compiler_params=pltpu.CompilerParams(needs_layout_passes=False),
    )
    def scatter_kernel(edges_hbm, h2_hbm, out_hbm, epk, sc0, sc1, dc0, dc1,
                       rb0, rb1, accum, sg0, sg1):
        cid = lax.axis_index("c")
        sid = lax.axis_index("s")
        wid = sid * 2 + cid
        pltpu.sync_copy(edges_hbm.at[wid], epk)

        zeros = jnp.zeros((L,), jnp.float32)

        def zrow(i, carry):
            r = i // (CH // L)
            k = i % (CH // L)
            rb0[r, pl.ds(k * L, L)] = zeros
            return carry

        lax.fori_loop(0, C * (CH // L), zrow, 0)
        rows_per_tile = N_PAD // 16
        base = sid * rows_per_tile

        def zslice(k, carry):
            pltpu.sync_copy(rb0, accum.at[pl.ds(base + k * C, C)])
            return carry

        lax.fori_loop(0, rows_per_tile // C, zslice, 0)
        plsc.subcore_barrier()

        def unpack(g, sc, dc):
            def upk(k, carry):
                w16 = epk[g, pl.ds(k * L, L)]
                sc[0, pl.ds(k * L, L)] = w16 & MASK
                dc[0, pl.ds(k * L, L)] = lax.shift_right_logical(w16, 16)
                return carry

            lax.fori_loop(0, C // L, upk, 0)

        unpack(0, sc0, dc0)
        pltpu.async_copy(h2_hbm.at[sc0.at[0]], rb0, sg0).wait()
        pltpu.async_copy(rb0, accum.at[dc0.at[0]], sg0, add=True)
        unpack(1, sc1, dc1)
        pltpu.async_copy(h2_hbm.at[sc1.at[0]], rb1, sg1).wait()
        pltpu.async_copy(rb1, accum.at[dc1.at[0]], sg1, add=True)

        def pair(p, carry):
            g = p * 2
            pltpu.make_async_copy(rb0, accum.at[pl.ds(0, C)], sg0).wait()
            unpack(g, sc0, dc0)
            pltpu.async_copy(h2_hbm.at[sc0.at[0]], rb0, sg0).wait()
            pltpu.async_copy(rb0, accum.at[dc0.at[0]], sg0, add=True)
            pltpu.make_async_copy(rb1, accum.at[pl.ds(0, C)], sg1).wait()
            unpack(g + 1, sc1, dc1)
            pltpu.async_copy(h2_hbm.at[sc1.at[0]], rb1, sg1).wait()
            pltpu.async_copy(rb1, accum.at[dc1.at[0]], sg1, add=True)
            return carry

        lax.fori_loop(1, nch // 2, pair, 0)
        pltpu.make_async_copy(rb0, accum.at[pl.ds(0, C)], sg0).wait()
        pltpu.make_async_copy(rb1, accum.at[pl.ds(0, C)], sg1).wait()
        plsc.subcore_barrier()
        pltpu.sync_copy(accum.at[pl.ds(base, rows_per_tile)],
                        out_hbm.at[cid, pl.ds(base, rows_per_tile)])

    return scatter_kernel


def _h2_body(x_ref, w_ref, degp_ref, h2_ref):
    deg = jnp.sum(degp_ref[...], axis=0)
    dis = lax.rsqrt(jnp.maximum(deg, 1.0))
    h = jnp.dot(x_ref[...], w_ref[...], preferred_element_type=jnp.float32)
    h2_ref[...] = h * dis[:, None]


_h2_kernel = pl.pallas_call(
    _h2_body,
    grid=(N_PAD // BLK,),
    in_specs=[
        pl.BlockSpec((BLK, CH), lambda i: (i, 0)),
        pl.BlockSpec((CH, CH), lambda i: (0, 0)),
        pl.BlockSpec((NW, BLK), lambda i: (0, i)),
    ],
    out_specs=pl.BlockSpec((BLK, CH), lambda i: (i, 0)),
    out_shape=jax.ShapeDtypeStruct((N_PAD, CH), jnp.float32),
)


def _out_body(p0_ref, p1_ref, degp_ref, b_ref, o_ref):
    deg = jnp.sum(degp_ref[...], axis=0)
    dis = lax.rsqrt(jnp.maximum(deg, 1.0))
    acc = (p0_ref[...] + p1_ref[...]) * dis[:, None]
    o_ref[...] = jnp.maximum(acc + b_ref[...], 0.0)


_out_kernel = pl.pallas_call(
    _out_body,
    grid=(N_PAD // BLK,),
    in_specs=[
        pl.BlockSpec((BLK, CH), lambda i: (i, 0)),
        pl.BlockSpec((BLK, CH), lambda i: (i, 0)),
        pl.BlockSpec((NW, BLK), lambda i: (0, i)),
        pl.BlockSpec((1, CH), lambda i: (0, 0)),
    ],
    out_specs=pl.BlockSpec((BLK, CH), lambda i: (i, 0)),
    out_shape=jax.ShapeDtypeStruct((N_PAD, CH), jnp.float32),
)


@jax.jit
def kernel(x, edge_index, W, b):
    n = x.shape[0]
    src = edge_index[0].astype(jnp.int32)
    dst = edge_index[1].astype(jnp.int32)
    loop = jnp.arange(n, dtype=jnp.int32)
    e_real = src.shape[0] + n
    ept = -(-e_real // (NW * 2 * C)) * 2 * C
    e_pad = ept * NW
    nch = ept // C
    pad = e_pad - e_real
    fill = jnp.full((pad,), n, jnp.int32)
    src_all = jnp.concatenate([src, loop, fill])
    dst_all = jnp.concatenate([dst, loop, fill])
    edges = jnp.bitwise_or(src_all, jnp.left_shift(dst_all, 16))
    x_pad = jnp.concatenate([x, jnp.zeros((N_PAD - n, CH), x.dtype)], axis=0)

    degp = _deg_call(nch)(edges.reshape(NW, nch * C))
    h2 = _h2_kernel(x_pad, W, degp)
    partials = _scatter_call(nch)(edges.reshape(NW, nch, C), h2)
    out = _out_kernel(partials[0], partials[1], degp, b.reshape(1, CH))
    return out[:n]


# speedup vs baseline: 1.6056x; 1.6056x over previous
import functools

import jax
import jax.numpy as jnp
from jax import lax
from jax.experimental import pallas as pl
from jax.experimental.pallas import tpu as pltpu
from jax.experimental.pallas import tpu_sc as plsc

CH = 128
N_PAD = 10240
NW = 32
C = 128
L = 16
BLK = 256


def _sc_mesh():
    return plsc.VectorSubcoreMesh(core_axis_name="c", subcore_axis_name="s")


@functools.lru_cache(maxsize=None)
def _deg_call(nch):
    @functools.partial(
        pl.kernel,
        out_type=jax.ShapeDtypeStruct((NW, N_PAD), jnp.float32),
        mesh=_sc_mesh(),
        scratch_types=[
            pltpu.VMEM((nch * C,), jnp.int32),
            pltpu.VMEM((N_PAD,), jnp.float32),
        ],
        compiler_params=pltpu.CompilerParams(needs_layout_passes=False),
    )
    def deg_kernel(dst_hbm, degp_hbm, dbuf, hist):
        cid = lax.axis_index("c")
        sid = lax.axis_index("s")
        wid = sid * 2 + cid
        pltpu.sync_copy(dst_hbm.at[wid], dbuf)
        zeros = jnp.zeros((L,), jnp.float32)
        ones = jnp.ones((L,), jnp.float32)

        def zero_body(i, carry):
            hist[pl.ds(i * L, L)] = zeros
            return carry

        lax.fori_loop(0, N_PAD // L, zero_body, 0)

        def vec_body(j, carry):
            d16 = dbuf[pl.ds(j * L, L)]
            plsc.addupdate_scatter(hist, [d16], ones)
            return carry

        lax.fori_loop(0, (nch * C) // L, vec_body, 0)
        pltpu.sync_copy(hist, degp_hbm.at[wid])

    return deg_kernel


@functools.lru_cache(maxsize=None)
def _scatter_call(nch):
    @functools.partial(
        pl.kernel,
        out_type=jax.ShapeDtypeStruct((2, N_PAD, CH), jnp.float32),
        mesh=_sc_mesh(),
        scratch_types=[
            pltpu.VMEM((nch, C), jnp.int32),
            pltpu.VMEM((nch, C), jnp.int32),
            pltpu.VMEM((C, CH), jnp.float32),
            pltpu.VMEM_SHARED((N_PAD, CH), jnp.float32),
            pltpu.SemaphoreType.DMA,
        ],
        compiler_params=pltpu.CompilerParams(needs_layout_passes=False),
    )
    def scatter_kernel(src_hbm, dst_hbm, h2_hbm, out_hbm, sidx, didx, rows,
                       accum, sem):
        cid = lax.axis_index("c")
        sid = lax.axis_index("s")
        wid = sid * 2 + cid
        pltpu.sync_copy(src_hbm.at[wid], sidx)
        pltpu.sync_copy(dst_hbm.at[wid], didx)
        zeros = jnp.zeros((L,), jnp.float32)

        def zrow(i, carry):
            r = i // (CH // L)
            k = i % (CH // L)
            rows[r, pl.ds(k * L, L)] = zeros
            return carry

        lax.fori_loop(0, C * (CH // L), zrow, 0)
        rows_per_tile = N_PAD // 16
        base = sid * rows_per_tile

        def zslice(k, carry):
            pltpu.sync_copy(rows, accum.at[pl.ds(base + k * C, C)])
            return carry

        lax.fori_loop(0, rows_per_tile // C, zslice, 0)
        plsc.subcore_barrier()

        def chunk(g, carry):
            pltpu.async_copy(h2_hbm.at[sidx.at[g]], rows, sem).wait()
            pltpu.sync_copy(rows, accum.at[didx.at[g]], add=True)
            return carry

        lax.fori_loop(0, nch, chunk, 0)
        plsc.subcore_barrier()
        pltpu.sync_copy(accum.at[pl.ds(base, rows_per_tile)],
                        out_hbm.at[cid, pl.ds(base, rows_per_tile)])

    return scatter_kernel


def _h2_body(x_ref, w_ref, degp_ref, h2_ref):
    deg = jnp.sum(degp_ref[...], axis=0)
    dis = lax.rsqrt(jnp.maximum(deg, 1.0))
    h = jnp.dot(x_ref[...], w_ref[...], preferred_element_type=jnp.float32)
    h2_ref[...] = h * dis[:, None]


_h2_kernel = pl.pallas_call(
    _h2_body,
    grid=(N_PAD // BLK,),
    in_specs=[
        pl.BlockSpec((BLK, CH), lambda i: (i, 0)),
        pl.BlockSpec((CH, CH), lambda i: (0, 0)),
        pl.BlockSpec((NW, BLK), lambda i: (0, i)),
    ],
    out_specs=pl.BlockSpec((BLK, CH), lambda i: (i, 0)),
    out_shape=jax.ShapeDtypeStruct((N_PAD, CH), jnp.float32),
)


def _out_body(p0_ref, p1_ref, degp_ref, b_ref, o_ref):
    deg = jnp.sum(degp_ref[...], axis=0)
    dis = lax.rsqrt(jnp.maximum(deg, 1.0))
    acc = (p0_ref[...] + p1_ref[...]) * dis[:, None]
    o_ref[...] = jnp.maximum(acc + b_ref[...], 0.0)


_out_kernel = pl.pallas_call(
    _out_body,
    grid=(N_PAD // BLK,),
    in_specs=[
        pl.BlockSpec((BLK, CH), lambda i: (i, 0)),
        pl.BlockSpec((BLK, CH), lambda i: (i, 0)),
        pl.BlockSpec((NW, BLK), lambda i: (0, i)),
        pl.BlockSpec((1, CH), lambda i: (0, 0)),
    ],
    out_specs=pl.BlockSpec((BLK, CH), lambda i: (i, 0)),
    out_shape=jax.ShapeDtypeStruct((N_PAD, CH), jnp.float32),
)


@jax.jit
def kernel(x, edge_index, W, b):
    n = x.shape[0]
    src = edge_index[0].astype(jnp.int32)
    dst = edge_index[1].astype(jnp.int32)
    loop = jnp.arange(n, dtype=jnp.int32)
    e_real = src.shape[0] + n
    ept = -(-e_real // (NW * C)) * C
    e_pad = ept * NW
    nch = ept // C
    pad = e_pad - e_real
    fill = jnp.full((pad,), n, jnp.int32)
    src_all = jnp.concatenate([src, loop, fill]).reshape(NW, nch, C)
    dst_all = jnp.concatenate([dst, loop, fill]).reshape(NW, nch, C)
    x_pad = jnp.concatenate([x, jnp.zeros((N_PAD - n, CH), x.dtype)], axis=0)

    degp = _deg_call(nch)(dst_all.reshape(NW, nch * C))
    h2 = _h2_kernel(x_pad, W, degp)
    partials = _scatter_call(nch)(src_all, dst_all, h2)
    out = _out_kernel(partials[0], partials[1], degp, b.reshape(1, CH))
    return out[:n]
